# Initial kernel scaffold; baseline (speedup 1.0000x reference)
import jax
import jax.numpy as jnp
from jax.experimental import pallas as pl


def kernel(node_type_ids, adj, node_emb, type_emb, W1, b1, W2, b2):
    raise NotImplementedError("write your pallas kernel here")


# speedup vs baseline: 1.1947x; 1.1947x over previous
import jax
import jax.numpy as jnp
from jax.experimental import pallas as pl
from jax.experimental.pallas import tpu as pltpu


def _pick_block(n, cap):
    best = 8
    for b in range(8, cap + 1, 8):
        if n % b == 0:
            best = b
    return best


def _encode_proj_body(ids_ref, emb_ref, temb_ref, w1_ref, out_ref):
    ids = ids_ref[...]
    nt = temb_ref.shape[0]
    onehot = (ids == jax.lax.broadcasted_iota(jnp.int32, (ids.shape[0], nt), 1))
    x = emb_ref[...] + jnp.dot(onehot.astype(jnp.float32), temb_ref[...],
                               preferred_element_type=jnp.float32)
    out_ref[...] = jnp.dot(x, w1_ref[...], preferred_element_type=jnp.float32)


def _spmm_relu_proj_body(adj_ref, y_ref, b1_ref, w2_ref, out_ref):
    t = jnp.dot(adj_ref[...], y_ref[...], preferred_element_type=jnp.float32)
    h = jnp.maximum(t + b1_ref[...], 0.0)
    out_ref[...] = jnp.dot(h, w2_ref[...], preferred_element_type=jnp.float32)


def _spmm_bias_body(adj_ref, y_ref, b2_ref, out_ref):
    out_ref[...] = jnp.dot(adj_ref[...], y_ref[...],
                           preferred_element_type=jnp.float32) + b2_ref[...]


def kernel(node_type_ids, adj, node_emb, type_emb, W1, b1, W2, b2):
    N, H = node_emb.shape
    E = W2.shape[1]
    T = type_emb.shape[0]
    ids2 = node_type_ids.reshape(N, 1)
    b1r = b1.reshape(1, H)
    b2r = b2.reshape(1, E)

    TB = _pick_block(N, 2048)
    y1 = pl.pallas_call(
        _encode_proj_body,
        grid=(N // TB,),
        in_specs=[
            pl.BlockSpec((TB, 1), lambda i: (i, 0)),
            pl.BlockSpec((TB, H), lambda i: (i, 0)),
            pl.BlockSpec((T, H), lambda i: (0, 0)),
            pl.BlockSpec((H, H), lambda i: (0, 0)),
        ],
        out_specs=pl.BlockSpec((TB, H), lambda i: (i, 0)),
        out_shape=jax.ShapeDtypeStruct((N, H), jnp.float32),
        compiler_params=pltpu.CompilerParams(
            dimension_semantics=("arbitrary",)),
    )(ids2, node_emb, type_emb, W1)

    TI = _pick_block(N, 512)
    y2 = pl.pallas_call(
        _spmm_relu_proj_body,
        grid=(N // TI,),
        in_specs=[
            pl.BlockSpec((TI, N), lambda i: (i, 0)),
            pl.BlockSpec((N, H), lambda i: (0, 0)),
            pl.BlockSpec((1, H), lambda i: (0, 0)),
            pl.BlockSpec((H, E), lambda i: (0, 0)),
        ],
        out_specs=pl.BlockSpec((TI, E), lambda i: (i, 0)),
        out_shape=jax.ShapeDtypeStruct((N, E), jnp.float32),
        compiler_params=pltpu.CompilerParams(
            dimension_semantics=("arbitrary",)),
    )(adj, y1, b1r, W2)

    z = pl.pallas_call(
        _spmm_bias_body,
        grid=(N // TI,),
        in_specs=[
            pl.BlockSpec((TI, N), lambda i: (i, 0)),
            pl.BlockSpec((N, E), lambda i: (0, 0)),
            pl.BlockSpec((1, E), lambda i: (0, 0)),
        ],
        out_specs=pl.BlockSpec((TI, E), lambda i: (i, 0)),
        out_shape=jax.ShapeDtypeStruct((N, E), jnp.float32),
        compiler_params=pltpu.CompilerParams(
            dimension_semantics=("arbitrary",)),
    )(adj, y2, b2r)
    return z
